# Initial kernel scaffold; baseline (speedup 1.0000x reference)
import jax
import jax.numpy as jnp
from jax.experimental import pallas as pl


def kernel(logit, target):
    raise NotImplementedError("write your pallas kernel here")


# speedup vs baseline: 200.3581x; 200.3581x over previous
import functools
import math

import jax
import jax.numpy as jnp
from jax import lax
from jax.experimental import pallas as pl
from jax.experimental.pallas import tpu as pltpu
from jax.experimental.pallas import tpu_sc as plsc

ALPHA = 0.25
SMOOTH = 1e-05

B = 2
C = 4
DHW = 128 * 128 * 128
N = B * DHW
NC, NS = 2, 16
NW = NC * NS
PERW = N // NW
K = 8192
NCHUNK = PERW // K
LANES = 16

C1 = 1.0 - SMOOTH - SMOOTH / 3.0
C2 = SMOOTH / 3.0
LN2 = math.log(2.0)
SQRT2 = math.sqrt(2.0)


def _focal_partials(logit_flat, target_flat):
    mesh = plsc.VectorSubcoreMesh(core_axis_name="c", subcore_axis_name="s",
                                  num_cores=NC, num_subcores=NS)

    @functools.partial(
        pl.kernel,
        out_type=jax.ShapeDtypeStruct((NW * LANES,), jnp.float32),
        mesh=mesh,
        compiler_params=pltpu.CompilerParams(needs_layout_passes=False),
        scratch_types=[
            pltpu.VMEM((C * K,), jnp.float32),
            pltpu.VMEM((C * K,), jnp.float32),
            pltpu.VMEM((K,), jnp.int32),
            pltpu.VMEM((K,), jnp.int32),
            pltpu.VMEM((LANES,), jnp.float32),
            pltpu.SemaphoreType.DMA,
            pltpu.SemaphoreType.DMA,
        ],
    )
    def k(l_hbm, t_hbm, out_hbm, lb0, lb1, tb0, tb1, accb, sem0, sem1):
        wid = lax.axis_index("c") * NS + lax.axis_index("s")
        b = wid // NS
        p0 = (wid % NS) * PERW
        lbufs = (lb0, lb1)
        tbufs = (tb0, tb1)
        sems = (sem0, sem1)

        def start(g):
            bank = g % 2
            cps = []
            for ch in range(C):
                off = (b * C + ch) * DHW + p0 + g * K
                cps.append(pltpu.async_copy(
                    l_hbm.at[pl.ds(off, K)],
                    lbufs[bank].at[pl.ds(ch * K, K)],
                    sems[bank]))
            cps.append(pltpu.async_copy(
                t_hbm.at[pl.ds(b * DHW + p0 + g * K, K)],
                tbufs[bank], sems[bank]))
            return cps

        iota = lax.iota(jnp.int32, LANES)

        def make_body(lb, tb):
            def body(i, acc):
                base = i * LANES
                t = tb[pl.ds(base, LANES)]
                l0 = lb[pl.ds(base, LANES)]
                l1 = lb[pl.ds(K + base, LANES)]
                l2 = lb[pl.ds(2 * K + base, LANES)]
                l3 = lb[pl.ds(3 * K + base, LANES)]
                lt = jnp.where(t == 0, l0,
                               jnp.where(t == 1, l1,
                                         jnp.where(t == 2, l2, l3)))
                s = (l0 + l1) + (l2 + l3)
                pt = C1 * lt + C2 * s + SMOOTH
                at = jnp.where(t == 0, ALPHA, 1.0 - ALPHA)
                bits = plsc.bitcast(pt, jnp.int32)
                e = (bits >> 23) - 127
                m = plsc.bitcast((bits & 0x007FFFFF) | 0x3F800000, jnp.float32)
                big = m >= SQRT2
                mm = jnp.where(big, m * 0.5, m)
                ef = e.astype(jnp.float32) + jnp.where(big, 1.0, 0.0)
                z = (mm - 1.0) / (mm + 1.0)
                z2 = z * z
                logpt = ef * LN2 + z * (
                    2.0 + z2 * (2.0 / 3.0 + z2 * (2.0 / 5.0 + z2 * (2.0 / 7.0))))
                omp = 1.0 - pt
                return acc - at * (omp * omp) * logpt
            return body

        acc = jnp.zeros((LANES,), jnp.float32)
        pending = {0: start(0)}
        for g in range(NCHUNK):
            if g + 1 < NCHUNK:
                pending[g + 1] = start(g + 1)
            for cp in pending.pop(g):
                cp.wait()
            acc = lax.fori_loop(0, K // LANES,
                                make_body(lbufs[g % 2], tbufs[g % 2]), acc)
        accb[...] = acc
        pltpu.sync_copy(accb, out_hbm.at[pl.ds(wid * LANES, LANES)])

    return k(logit_flat, target_flat)


def kernel(logit, target):
    partials = _focal_partials(logit.reshape(-1), target.reshape(-1))
    return jnp.sum(partials) / N


# speedup vs baseline: 242.7971x; 1.2118x over previous
import functools
import math

import jax
import jax.numpy as jnp
from jax import lax
from jax.experimental import pallas as pl
from jax.experimental.pallas import tpu as pltpu
from jax.experimental.pallas import tpu_sc as plsc

ALPHA = 0.25
SMOOTH = 1e-05

B = 2
C = 4
DHW = 128 * 128 * 128
N = B * DHW
NC, NS = 2, 16
NW = NC * NS
PERW = N // NW
K = 8192
NCHUNK = PERW // K
LANES = 16

C1 = 1.0 - SMOOTH - SMOOTH / 3.0
C2 = SMOOTH / 3.0
LN2 = math.log(2.0)
SQRT2 = math.sqrt(2.0)


def _focal_partials(logit_flat, target_flat):
    mesh = plsc.VectorSubcoreMesh(core_axis_name="c", subcore_axis_name="s",
                                  num_cores=NC, num_subcores=NS)

    @functools.partial(
        pl.kernel,
        out_type=jax.ShapeDtypeStruct((NW * LANES,), jnp.float32),
        mesh=mesh,
        compiler_params=pltpu.CompilerParams(needs_layout_passes=False),
        scratch_types=[
            pltpu.VMEM((C * K,), jnp.float32),
            pltpu.VMEM((C * K,), jnp.float32),
            pltpu.VMEM((K,), jnp.int32),
            pltpu.VMEM((K,), jnp.int32),
            pltpu.VMEM((LANES,), jnp.float32),
            pltpu.SemaphoreType.DMA,
            pltpu.SemaphoreType.DMA,
        ],
    )
    def k(l_hbm, t_hbm, out_hbm, lb0, lb1, tb0, tb1, accb, sem0, sem1):
        wid = lax.axis_index("c") * NS + lax.axis_index("s")
        b = wid // NS
        p0 = (wid % NS) * PERW
        lbufs = (lb0, lb1)
        tbufs = (tb0, tb1)
        sems = (sem0, sem1)

        def start(g):
            bank = g % 2
            cps = []
            for ch in range(C):
                off = (b * C + ch) * DHW + p0 + g * K
                cps.append(pltpu.async_copy(
                    l_hbm.at[pl.ds(off, K)],
                    lbufs[bank].at[pl.ds(ch * K, K)],
                    sems[bank]))
            cps.append(pltpu.async_copy(
                t_hbm.at[pl.ds(b * DHW + p0 + g * K, K)],
                tbufs[bank], sems[bank]))
            return cps

        iota = lax.iota(jnp.int32, LANES)

        def make_body(lb, tb):
            def body(i, acc):
                base = i * LANES
                t = tb[pl.ds(base, LANES)]
                l0 = lb[pl.ds(base, LANES)]
                l1 = lb[pl.ds(K + base, LANES)]
                l2 = lb[pl.ds(2 * K + base, LANES)]
                l3 = lb[pl.ds(3 * K + base, LANES)]
                lt = plsc.load_gather(lb, [t * K + (base + iota)])
                s = (l0 + l1) + (l2 + l3)
                pt = C1 * lt + C2 * s + SMOOTH
                at = jnp.where(t == 0, ALPHA, 1.0 - ALPHA)
                bits = plsc.bitcast(pt, jnp.int32)
                ef = ((bits >> 23) - 127).astype(jnp.float32)
                m = plsc.bitcast((bits & 0x007FFFFF) | 0x3F800000, jnp.float32)
                z = (m - 1.0) / (m + 1.0)
                z2 = z * z
                logpt = ef * LN2 + z * (
                    2.0 + z2 * (2.0 / 3.0 + z2 * (2.0 / 5.0 + z2 * (2.0 / 7.0))))
                omp = 1.0 - pt
                return acc - at * (omp * omp) * logpt
            return body

        acc = jnp.zeros((LANES,), jnp.float32)
        pending = {0: start(0)}
        for g in range(NCHUNK):
            if g + 1 < NCHUNK:
                pending[g + 1] = start(g + 1)
            for cp in pending.pop(g):
                cp.wait()
            acc = lax.fori_loop(0, K // LANES,
                                make_body(lbufs[g % 2], tbufs[g % 2]), acc)
        accb[...] = acc
        pltpu.sync_copy(accb, out_hbm.at[pl.ds(wid * LANES, LANES)])

    return k(logit_flat, target_flat)


def kernel(logit, target):
    partials = _focal_partials(logit.reshape(-1), target.reshape(-1))
    return jnp.sum(partials) / N
